# baseline (speedup 1.0000x reference)
import functools

import jax
import jax.numpy as jnp
from jax import lax
from jax.experimental import pallas as pl
from jax.experimental.pallas import tpu as pltpu
from jax.experimental.pallas import tpu_sc as plsc

N = 10000
E = 320000
D = 128

NC = 2
NS = 16
NW = NC * NS

CH = 120
EPW = 10560
EP = NW * EPW
NCHUNK = EPW // CH

NP = 10112
RPT = NP // NS
DUMMY = N

BN = 1000

f32 = jnp.float32


GROUP = 8
NGROUP = NCHUNK // GROUP


def _zero_spmem_slab(rows, acc, base):
    z16 = jnp.zeros((16,), f32)

    def zrow(i, _):
        rows[i // 8, pl.ds((i % 8) * 16, 16)] = z16
        return 0
    lax.fori_loop(0, CH * 8, zrow, 0)
    off = 0
    while off < RPT:
        nr = min(CH, RPT - off)
        pltpu.sync_copy(rows.at[pl.ds(0, nr)], acc.at[pl.ds(base + off, nr)])
        off += nr


def _writeout_slab(rows, acc, out_hbm, cid, base):
    off = 0
    while off < RPT:
        nr = min(CH, RPT - off)
        pltpu.sync_copy(acc.at[pl.ds(base + off, nr)], rows.at[pl.ds(0, nr)])
        pltpu.sync_copy(rows.at[pl.ds(0, nr)], out_hbm.at[cid, pl.ds(base + off, nr)])
        off += nr


HROWS = 624


def _make_gather():
    mesh = plsc.VectorSubcoreMesh(core_axis_name="c", subcore_axis_name="s")
    out_type = jax.ShapeDtypeStruct((NW, EPW, D), f32)
    scratch = [
        pltpu.VMEM((GROUP, CH), jnp.int32),
        pltpu.VMEM((CH, D), f32),
        pltpu.VMEM((CH, D), f32),
        pltpu.VMEM_SHARED((N, D), f32),
        pltpu.SemaphoreType.DMA,
        pltpu.SemaphoreType.DMA,
    ]

    def body(h_hbm, src_hbm, msgs_hbm, sidx, rows0, rows1, hsp, sem0, sem1):
        cid = lax.axis_index("c")
        sid = lax.axis_index("s")
        wid = sid * NC + cid
        bufs = (rows0, rows1)
        sems = (sem0, sem1)

        pltpu.sync_copy(h_hbm.at[pl.ds(sid * HROWS, HROWS)],
                        hsp.at[pl.ds(sid * HROWS, HROWS)])

        @pl.when(sid == 0)
        def _():
            pltpu.sync_copy(h_hbm.at[pl.ds(NS * HROWS, N - NS * HROWS)],
                            hsp.at[pl.ds(NS * HROWS, N - NS * HROWS)])
        plsc.subcore_barrier()

        def group(g, _):
            pltpu.sync_copy(src_hbm.at[wid, pl.ds(g * GROUP, GROUP)], sidx)
            gd = [None, None]
            gd[0] = pltpu.async_copy(hsp.at[sidx.at[0]], bufs[0], sems[0])
            for j in range(GROUP):
                b = j % 2
                if j + 1 < GROUP:
                    nb = (j + 1) % 2
                    gd[nb] = pltpu.async_copy(
                        hsp.at[sidx.at[j + 1]], bufs[nb], sems[nb])
                gd[b].wait()
                pltpu.sync_copy(
                    bufs[b], msgs_hbm.at[wid, pl.ds((g * GROUP + j) * CH, CH)])
            return 0
        lax.fori_loop(0, NGROUP, group, 0)

    return functools.partial(pl.kernel, out_type=out_type, mesh=mesh,
                             scratch_types=scratch)(body)


def _make_scatter():
    mesh = plsc.VectorSubcoreMesh(core_axis_name="c", subcore_axis_name="s")
    out_type = jax.ShapeDtypeStruct((NC, NP, D), f32)
    scratch = [
        pltpu.VMEM((GROUP, CH), jnp.int32),
        pltpu.VMEM((CH, D), f32),
        pltpu.VMEM((CH, D), f32),
        pltpu.VMEM_SHARED((NP, D), f32),
        pltpu.SemaphoreType.DMA,
        pltpu.SemaphoreType.DMA,
    ]

    def body(msgs_hbm, dst_hbm, out_hbm, didx, rows0, rows1, acc, sem0, sem1):
        cid = lax.axis_index("c")
        sid = lax.axis_index("s")
        wid = sid * NC + cid
        base = sid * RPT
        bufs = (rows0, rows1)
        sems = (sem0, sem1)

        _zero_spmem_slab(rows0, acc, base)
        plsc.subcore_barrier()

        def group(g, _):
            pltpu.sync_copy(dst_hbm.at[wid, pl.ds(g * GROUP, GROUP)], didx)
            rd = [None, None]
            rd[0] = pltpu.async_copy(
                msgs_hbm.at[wid, pl.ds(g * GROUP * CH, CH)], bufs[0], sems[0])
            for j in range(GROUP):
                b = j % 2
                if j + 1 < GROUP:
                    nb = (j + 1) % 2
                    rd[nb] = pltpu.async_copy(
                        msgs_hbm.at[wid, pl.ds((g * GROUP + j + 1) * CH, CH)],
                        bufs[nb], sems[nb])
                rd[b].wait()
                pltpu.sync_copy(bufs[b], acc.at[didx.at[j]], add=True)
            return 0
        lax.fori_loop(0, NGROUP, group, 0)

        plsc.subcore_barrier()
        _writeout_slab(rows0, acc, out_hbm, cid, base)

    return functools.partial(pl.kernel, out_type=out_type, mesh=mesh,
                             scratch_types=scratch)(body)


_gather_msgs = _make_gather()
_scatter_msgs = _make_scatter()


def _agg(h, srcp, dstp):
    return _scatter_msgs(_gather_msgs(h, srcp), dstp)


def _make_deg():
    mesh = plsc.VectorSubcoreMesh(core_axis_name="c", subcore_axis_name="s")
    out_type = jax.ShapeDtypeStruct((NC, NP, D), f32)
    scratch = [
        pltpu.VMEM((GROUP, CH), jnp.int32),
        pltpu.VMEM((CH, D), f32),
        pltpu.VMEM((CH, D), f32),
        pltpu.VMEM_SHARED((NP, D), f32),
    ]

    def body(dst_hbm, out_hbm, didx, ones, zbuf, acc):
        cid = lax.axis_index("c")
        sid = lax.axis_index("s")
        wid = sid * NC + cid
        base = sid * RPT

        _zero_spmem_slab(zbuf, acc, base)
        one16 = jnp.full((16,), 1.0, f32)

        def orow(i, _):
            ones[i // 8, pl.ds((i % 8) * 16, 16)] = one16
            return 0
        lax.fori_loop(0, CH * 8, orow, 0)
        plsc.subcore_barrier()

        def group(g, _):
            pltpu.sync_copy(dst_hbm.at[wid, pl.ds(g * GROUP, GROUP)], didx)

            def chunk(j, _):
                pltpu.sync_copy(ones, acc.at[didx.at[j]], add=True)
                return 0
            lax.fori_loop(0, GROUP, chunk, 0)
            return 0
        lax.fori_loop(0, NGROUP, group, 0)

        plsc.subcore_barrier()
        _writeout_slab(zbuf, acc, out_hbm, cid, base)

    return functools.partial(pl.kernel, out_type=out_type, mesh=mesh,
                             scratch_types=scratch)(body)


_deg = _make_deg()


def _inproj_body(x_ref, w_ref, b_ref, inp_ref, h_ref):
    acc = jnp.dot(x_ref[...], w_ref[...], preferred_element_type=f32) + b_ref[...]
    inp_ref[...] = acc
    h_ref[...] = jnp.maximum(acc, 0.0)


def _inproj(x, W, b):
    return pl.pallas_call(
        _inproj_body,
        grid=(N // BN,),
        in_specs=[
            pl.BlockSpec((BN, D), lambda i: (i, 0)),
            pl.BlockSpec((D, D), lambda i: (0, 0)),
            pl.BlockSpec((1, D), lambda i: (0, 0)),
        ],
        out_specs=[
            pl.BlockSpec((BN, D), lambda i: (i, 0)),
            pl.BlockSpec((BN, D), lambda i: (i, 0)),
        ],
        out_shape=[
            jax.ShapeDtypeStruct((N, D), f32),
            jax.ShapeDtypeStruct((N, D), f32),
        ],
    )(x, W, b.reshape(1, D))


def _conv_body(p0_ref, p1_ref, d0_ref, d1_ref, h_ref, inp_ref, wl_ref, bl_ref,
               wr_ref, out_ref, *, last):
    deg = d0_ref[0][:, 0:1] + d1_ref[0][:, 0:1]
    recip = 1.0 / jnp.maximum(deg, 1.0)
    agg = (p0_ref[0] + p1_ref[0]) * recip
    acc = (jnp.dot(agg, wl_ref[...], preferred_element_type=f32) + bl_ref[...]
           + jnp.dot(h_ref[...], wr_ref[...], preferred_element_type=f32))
    if last:
        m = jnp.max(acc, axis=-1, keepdims=True)
        s = acc - m
        out_ref[...] = s - jnp.log(jnp.sum(jnp.exp(s), axis=-1, keepdims=True))
    else:
        out_ref[...] = jnp.maximum(acc, 0.0) + 0.2 * inp_ref[...]


def _conv(parts, deg, h, inp, Wl, bl, Wr, last):
    return pl.pallas_call(
        functools.partial(_conv_body, last=last),
        grid=(N // BN,),
        in_specs=[
            pl.BlockSpec((1, BN, D), lambda i: (0, i, 0)),
            pl.BlockSpec((1, BN, D), lambda i: (1, i, 0)),
            pl.BlockSpec((1, BN, D), lambda i: (0, i, 0)),
            pl.BlockSpec((1, BN, D), lambda i: (1, i, 0)),
            pl.BlockSpec((BN, D), lambda i: (i, 0)),
            pl.BlockSpec((BN, D), lambda i: (i, 0)),
            pl.BlockSpec((D, D), lambda i: (0, 0)),
            pl.BlockSpec((1, D), lambda i: (0, 0)),
            pl.BlockSpec((D, D), lambda i: (0, 0)),
        ],
        out_specs=pl.BlockSpec((BN, D), lambda i: (i, 0)),
        out_shape=jax.ShapeDtypeStruct((N, D), f32),
    )(parts, parts, deg, deg, h, inp, Wl, bl.reshape(1, D), Wr)


def kernel(x, edge_index, W_in, b_in, Wl0, bl0, Wr0, Wl1, bl1, Wr1, Wl2, bl2, Wr2):
    src = edge_index[0].astype(jnp.int32)
    dst = edge_index[1].astype(jnp.int32)
    srcp = jnp.concatenate([src, jnp.zeros((EP - E,), jnp.int32)])
    dstp = jnp.concatenate([dst, jnp.full((EP - E,), DUMMY, jnp.int32)])
    srcp = srcp.reshape(NW, NCHUNK, CH)
    dstp = dstp.reshape(NW, NCHUNK, CH)

    inp, h0 = _inproj(x, W_in, b_in)
    degp = _deg(dstp)
    parts0 = _agg(h0, srcp, dstp)
    h1 = _conv(parts0, degp, h0, inp, Wl0, bl0, Wr0, last=False)
    parts1 = _agg(h1, srcp, dstp)
    h2 = _conv(parts1, degp, h1, inp, Wl1, bl1, Wr1, last=False)
    parts2 = _agg(h2, srcp, dstp)
    return _conv(parts2, degp, h2, inp, Wl2, bl2, Wr2, last=True)


# speedup vs baseline: 1.0919x; 1.0919x over previous
import functools

import jax
import jax.numpy as jnp
from jax import lax
from jax.experimental import pallas as pl
from jax.experimental.pallas import tpu as pltpu
from jax.experimental.pallas import tpu_sc as plsc

N = 10000
E = 320000
D = 128

NC = 2
NS = 16
NW = NC * NS

CH = 112
EPW = 10752
EP = NW * EPW
NCHUNK = EPW // CH

NP = 10112
RPT = NP // NS
DUMMY = N

BN = 1000

f32 = jnp.float32


GROUP = 32
NGROUP = NCHUNK // GROUP


def _zero_spmem_slab(rows, acc, base):
    z16 = jnp.zeros((16,), f32)

    def zrow(i, _):
        rows[i // 8, pl.ds((i % 8) * 16, 16)] = z16
        return 0
    lax.fori_loop(0, CH * 8, zrow, 0)
    off = 0
    while off < RPT:
        nr = min(CH, RPT - off)
        pltpu.sync_copy(rows.at[pl.ds(0, nr)], acc.at[pl.ds(base + off, nr)])
        off += nr


def _writeout_slab(rows, acc, out_hbm, cid, base):
    off = 0
    while off < RPT:
        nr = min(CH, RPT - off)
        pltpu.sync_copy(acc.at[pl.ds(base + off, nr)], rows.at[pl.ds(0, nr)])
        pltpu.sync_copy(rows.at[pl.ds(0, nr)], out_hbm.at[cid, pl.ds(base + off, nr)])
        off += nr


HROWS = 624


def _make_gather():
    mesh = plsc.VectorSubcoreMesh(core_axis_name="c", subcore_axis_name="s")
    out_type = jax.ShapeDtypeStruct((NW, EPW, D), f32)
    scratch = [
        pltpu.VMEM((GROUP, CH), jnp.int32),
        pltpu.VMEM((CH, D), f32),
        pltpu.VMEM((CH, D), f32),
        pltpu.VMEM_SHARED((N, D), f32),
        pltpu.SemaphoreType.DMA,
        pltpu.SemaphoreType.DMA,
    ]

    def body(h_hbm, src_hbm, msgs_hbm, sidx, rows0, rows1, hsp, sem0, sem1):
        cid = lax.axis_index("c")
        sid = lax.axis_index("s")
        wid = sid * NC + cid
        bufs = (rows0, rows1)
        sems = (sem0, sem1)

        pltpu.sync_copy(h_hbm.at[pl.ds(sid * HROWS, HROWS)],
                        hsp.at[pl.ds(sid * HROWS, HROWS)])

        @pl.when(sid == 0)
        def _():
            pltpu.sync_copy(h_hbm.at[pl.ds(NS * HROWS, N - NS * HROWS)],
                            hsp.at[pl.ds(NS * HROWS, N - NS * HROWS)])
        plsc.subcore_barrier()

        def group(g, _):
            pltpu.sync_copy(src_hbm.at[wid, pl.ds(g * GROUP, GROUP)], sidx)
            gd = [None, None]
            gd[0] = pltpu.async_copy(hsp.at[sidx.at[0]], bufs[0], sems[0])
            for j in range(GROUP):
                b = j % 2
                if j + 1 < GROUP:
                    nb = (j + 1) % 2
                    gd[nb] = pltpu.async_copy(
                        hsp.at[sidx.at[j + 1]], bufs[nb], sems[nb])
                gd[b].wait()
                pltpu.sync_copy(
                    bufs[b], msgs_hbm.at[wid, pl.ds((g * GROUP + j) * CH, CH)])
            return 0
        lax.fori_loop(0, NGROUP, group, 0)

    return functools.partial(pl.kernel, out_type=out_type, mesh=mesh,
                             scratch_types=scratch)(body)


def _make_scatter():
    mesh = plsc.VectorSubcoreMesh(core_axis_name="c", subcore_axis_name="s")
    out_type = jax.ShapeDtypeStruct((NC, NP, D), f32)
    scratch = [
        pltpu.VMEM((GROUP, CH), jnp.int32),
        pltpu.VMEM((CH, D), f32),
        pltpu.VMEM((CH, D), f32),
        pltpu.VMEM_SHARED((NP, D), f32),
        pltpu.SemaphoreType.DMA,
        pltpu.SemaphoreType.DMA,
    ]

    def body(msgs_hbm, dst_hbm, out_hbm, didx, rows0, rows1, acc, sem0, sem1):
        cid = lax.axis_index("c")
        sid = lax.axis_index("s")
        wid = sid * NC + cid
        base = sid * RPT
        bufs = (rows0, rows1)
        sems = (sem0, sem1)

        _zero_spmem_slab(rows0, acc, base)
        plsc.subcore_barrier()

        def group(g, _):
            pltpu.sync_copy(dst_hbm.at[wid, pl.ds(g * GROUP, GROUP)], didx)
            rd = [None, None]
            rd[0] = pltpu.async_copy(
                msgs_hbm.at[wid, pl.ds(g * GROUP * CH, CH)], bufs[0], sems[0])
            for j in range(GROUP):
                b = j % 2
                if j + 1 < GROUP:
                    nb = (j + 1) % 2
                    rd[nb] = pltpu.async_copy(
                        msgs_hbm.at[wid, pl.ds((g * GROUP + j + 1) * CH, CH)],
                        bufs[nb], sems[nb])
                rd[b].wait()
                pltpu.sync_copy(bufs[b], acc.at[didx.at[j]], add=True)
            return 0
        lax.fori_loop(0, NGROUP, group, 0)

        plsc.subcore_barrier()
        _writeout_slab(rows0, acc, out_hbm, cid, base)

    return functools.partial(pl.kernel, out_type=out_type, mesh=mesh,
                             scratch_types=scratch)(body)


_gather_msgs = _make_gather()
_scatter_msgs = _make_scatter()


def _agg(h, srcp, dstp):
    return _scatter_msgs(_gather_msgs(h, srcp), dstp)


def _make_deg():
    mesh = plsc.VectorSubcoreMesh(core_axis_name="c", subcore_axis_name="s")
    out_type = jax.ShapeDtypeStruct((NC, NP, D), f32)
    scratch = [
        pltpu.VMEM((GROUP, CH), jnp.int32),
        pltpu.VMEM((CH, D), f32),
        pltpu.VMEM((CH, D), f32),
        pltpu.VMEM_SHARED((NP, D), f32),
    ]

    def body(dst_hbm, out_hbm, didx, ones, zbuf, acc):
        cid = lax.axis_index("c")
        sid = lax.axis_index("s")
        wid = sid * NC + cid
        base = sid * RPT

        _zero_spmem_slab(zbuf, acc, base)
        one16 = jnp.full((16,), 1.0, f32)

        def orow(i, _):
            ones[i // 8, pl.ds((i % 8) * 16, 16)] = one16
            return 0
        lax.fori_loop(0, CH * 8, orow, 0)
        plsc.subcore_barrier()

        def group(g, _):
            pltpu.sync_copy(dst_hbm.at[wid, pl.ds(g * GROUP, GROUP)], didx)

            def chunk(j, _):
                pltpu.sync_copy(ones, acc.at[didx.at[j]], add=True)
                return 0
            lax.fori_loop(0, GROUP, chunk, 0)
            return 0
        lax.fori_loop(0, NGROUP, group, 0)

        plsc.subcore_barrier()
        _writeout_slab(zbuf, acc, out_hbm, cid, base)

    return functools.partial(pl.kernel, out_type=out_type, mesh=mesh,
                             scratch_types=scratch)(body)


_deg = _make_deg()


def _inproj_body(x_ref, w_ref, b_ref, inp_ref, h_ref):
    acc = jnp.dot(x_ref[...], w_ref[...], preferred_element_type=f32) + b_ref[...]
    inp_ref[...] = acc
    h_ref[...] = jnp.maximum(acc, 0.0)


def _inproj(x, W, b):
    return pl.pallas_call(
        _inproj_body,
        grid=(N // BN,),
        in_specs=[
            pl.BlockSpec((BN, D), lambda i: (i, 0)),
            pl.BlockSpec((D, D), lambda i: (0, 0)),
            pl.BlockSpec((1, D), lambda i: (0, 0)),
        ],
        out_specs=[
            pl.BlockSpec((BN, D), lambda i: (i, 0)),
            pl.BlockSpec((BN, D), lambda i: (i, 0)),
        ],
        out_shape=[
            jax.ShapeDtypeStruct((N, D), f32),
            jax.ShapeDtypeStruct((N, D), f32),
        ],
    )(x, W, b.reshape(1, D))


def _conv_body(p0_ref, p1_ref, d0_ref, d1_ref, h_ref, inp_ref, wl_ref, bl_ref,
               wr_ref, out_ref, *, last):
    deg = d0_ref[0][:, 0:1] + d1_ref[0][:, 0:1]
    recip = 1.0 / jnp.maximum(deg, 1.0)
    agg = (p0_ref[0] + p1_ref[0]) * recip
    acc = (jnp.dot(agg, wl_ref[...], preferred_element_type=f32) + bl_ref[...]
           + jnp.dot(h_ref[...], wr_ref[...], preferred_element_type=f32))
    if last:
        m = jnp.max(acc, axis=-1, keepdims=True)
        s = acc - m
        out_ref[...] = s - jnp.log(jnp.sum(jnp.exp(s), axis=-1, keepdims=True))
    else:
        out_ref[...] = jnp.maximum(acc, 0.0) + 0.2 * inp_ref[...]


def _conv(parts, deg, h, inp, Wl, bl, Wr, last):
    return pl.pallas_call(
        functools.partial(_conv_body, last=last),
        grid=(N // BN,),
        in_specs=[
            pl.BlockSpec((1, BN, D), lambda i: (0, i, 0)),
            pl.BlockSpec((1, BN, D), lambda i: (1, i, 0)),
            pl.BlockSpec((1, BN, D), lambda i: (0, i, 0)),
            pl.BlockSpec((1, BN, D), lambda i: (1, i, 0)),
            pl.BlockSpec((BN, D), lambda i: (i, 0)),
            pl.BlockSpec((BN, D), lambda i: (i, 0)),
            pl.BlockSpec((D, D), lambda i: (0, 0)),
            pl.BlockSpec((1, D), lambda i: (0, 0)),
            pl.BlockSpec((D, D), lambda i: (0, 0)),
        ],
        out_specs=pl.BlockSpec((BN, D), lambda i: (i, 0)),
        out_shape=jax.ShapeDtypeStruct((N, D), f32),
    )(parts, parts, deg, deg, h, inp, Wl, bl.reshape(1, D), Wr)


def kernel(x, edge_index, W_in, b_in, Wl0, bl0, Wr0, Wl1, bl1, Wr1, Wl2, bl2, Wr2):
    src = edge_index[0].astype(jnp.int32)
    dst = edge_index[1].astype(jnp.int32)
    srcp = jnp.concatenate([src, jnp.zeros((EP - E,), jnp.int32)])
    dstp = jnp.concatenate([dst, jnp.full((EP - E,), DUMMY, jnp.int32)])
    srcp = srcp.reshape(NW, NCHUNK, CH)
    dstp = dstp.reshape(NW, NCHUNK, CH)

    inp, h0 = _inproj(x, W_in, b_in)
    degp = _deg(dstp)
    parts0 = _agg(h0, srcp, dstp)
    h1 = _conv(parts0, degp, h0, inp, Wl0, bl0, Wr0, last=False)
    parts1 = _agg(h1, srcp, dstp)
    h2 = _conv(parts1, degp, h1, inp, Wl1, bl1, Wr1, last=False)
    parts2 = _agg(h2, srcp, dstp)
    return _conv(parts2, degp, h2, inp, Wl2, bl2, Wr2, last=True)


# speedup vs baseline: 1.1114x; 1.0179x over previous
import functools

import jax
import jax.numpy as jnp
from jax import lax
from jax.experimental import pallas as pl
from jax.experimental.pallas import tpu as pltpu
from jax.experimental.pallas import tpu_sc as plsc

N = 10000
E = 320000
D = 128

NC = 2
NS = 16
NW = NC * NS

CH = 112
EPW = 10752
EP = NW * EPW
NCHUNK = EPW // CH

NP = 10112
RPT = NP // NS
DUMMY = N

BN = 1000

f32 = jnp.float32


GROUP = 48
NGROUP = NCHUNK // GROUP


def _zero_spmem_slab(rows, acc, base):
    z16 = jnp.zeros((16,), f32)

    def zrow(i, _):
        rows[i // 8, pl.ds((i % 8) * 16, 16)] = z16
        return 0
    lax.fori_loop(0, CH * 8, zrow, 0)
    off = 0
    while off < RPT:
        nr = min(CH, RPT - off)
        pltpu.sync_copy(rows.at[pl.ds(0, nr)], acc.at[pl.ds(base + off, nr)])
        off += nr


def _writeout_slab(rows, acc, out_hbm, cid, base):
    off = 0
    while off < RPT:
        nr = min(CH, RPT - off)
        pltpu.sync_copy(acc.at[pl.ds(base + off, nr)], rows.at[pl.ds(0, nr)])
        pltpu.sync_copy(rows.at[pl.ds(0, nr)], out_hbm.at[cid, pl.ds(base + off, nr)])
        off += nr


HROWS = 624


def _make_gather():
    mesh = plsc.VectorSubcoreMesh(core_axis_name="c", subcore_axis_name="s")
    out_type = jax.ShapeDtypeStruct((NW, EPW, D), f32)
    scratch = [
        pltpu.VMEM((GROUP, CH), jnp.int32),
        pltpu.VMEM((CH, D), f32),
        pltpu.VMEM((CH, D), f32),
        pltpu.VMEM_SHARED((N, D), f32),
        pltpu.SemaphoreType.DMA,
        pltpu.SemaphoreType.DMA,
    ]

    def body(h_hbm, src_hbm, msgs_hbm, sidx, rows0, rows1, hsp, sem0, sem1):
        cid = lax.axis_index("c")
        sid = lax.axis_index("s")
        wid = sid * NC + cid
        bufs = (rows0, rows1)
        sems = (sem0, sem1)

        pltpu.sync_copy(h_hbm.at[pl.ds(sid * HROWS, HROWS)],
                        hsp.at[pl.ds(sid * HROWS, HROWS)])

        @pl.when(sid == 0)
        def _():
            pltpu.sync_copy(h_hbm.at[pl.ds(NS * HROWS, N - NS * HROWS)],
                            hsp.at[pl.ds(NS * HROWS, N - NS * HROWS)])
        plsc.subcore_barrier()

        def group(g, _):
            pltpu.sync_copy(src_hbm.at[wid, pl.ds(g * GROUP, GROUP)], sidx)
            gd = [None, None]
            gd[0] = pltpu.async_copy(hsp.at[sidx.at[0]], bufs[0], sems[0])
            for j in range(GROUP):
                b = j % 2
                if j + 1 < GROUP:
                    nb = (j + 1) % 2
                    gd[nb] = pltpu.async_copy(
                        hsp.at[sidx.at[j + 1]], bufs[nb], sems[nb])
                gd[b].wait()
                pltpu.sync_copy(
                    bufs[b], msgs_hbm.at[wid, pl.ds((g * GROUP + j) * CH, CH)])
            return 0
        lax.fori_loop(0, NGROUP, group, 0)

    return functools.partial(pl.kernel, out_type=out_type, mesh=mesh,
                             scratch_types=scratch)(body)


def _make_scatter():
    mesh = plsc.VectorSubcoreMesh(core_axis_name="c", subcore_axis_name="s")
    out_type = jax.ShapeDtypeStruct((NC, NP, D), f32)
    scratch = [
        pltpu.VMEM((GROUP, CH), jnp.int32),
        pltpu.VMEM((CH, D), f32),
        pltpu.VMEM((CH, D), f32),
        pltpu.VMEM_SHARED((NP, D), f32),
        pltpu.SemaphoreType.DMA,
        pltpu.SemaphoreType.DMA,
    ]

    def body(msgs_hbm, dst_hbm, out_hbm, didx, rows0, rows1, acc, sem0, sem1):
        cid = lax.axis_index("c")
        sid = lax.axis_index("s")
        wid = sid * NC + cid
        base = sid * RPT
        bufs = (rows0, rows1)
        sems = (sem0, sem1)

        _zero_spmem_slab(rows0, acc, base)
        plsc.subcore_barrier()

        def group(g, _):
            pltpu.sync_copy(dst_hbm.at[wid, pl.ds(g * GROUP, GROUP)], didx)
            rd = [None, None]
            rd[0] = pltpu.async_copy(
                msgs_hbm.at[wid, pl.ds(g * GROUP * CH, CH)], bufs[0], sems[0])
            for j in range(GROUP):
                b = j % 2
                if j + 1 < GROUP:
                    nb = (j + 1) % 2
                    rd[nb] = pltpu.async_copy(
                        msgs_hbm.at[wid, pl.ds((g * GROUP + j + 1) * CH, CH)],
                        bufs[nb], sems[nb])
                rd[b].wait()
                pltpu.sync_copy(bufs[b], acc.at[didx.at[j]], add=True)
            return 0
        lax.fori_loop(0, NGROUP, group, 0)

        plsc.subcore_barrier()
        _writeout_slab(rows0, acc, out_hbm, cid, base)

    return functools.partial(pl.kernel, out_type=out_type, mesh=mesh,
                             scratch_types=scratch)(body)


_gather_msgs = _make_gather()
_scatter_msgs = _make_scatter()


def _agg(h, srcp, dstp):
    return _scatter_msgs(_gather_msgs(h, srcp), dstp)


def _make_deg():
    mesh = plsc.VectorSubcoreMesh(core_axis_name="c", subcore_axis_name="s")
    out_type = jax.ShapeDtypeStruct((NC, NP, D), f32)
    scratch = [
        pltpu.VMEM((GROUP, CH), jnp.int32),
        pltpu.VMEM((CH, D), f32),
        pltpu.VMEM((CH, D), f32),
        pltpu.VMEM_SHARED((NP, D), f32),
    ]

    def body(dst_hbm, out_hbm, didx, ones, zbuf, acc):
        cid = lax.axis_index("c")
        sid = lax.axis_index("s")
        wid = sid * NC + cid
        base = sid * RPT

        _zero_spmem_slab(zbuf, acc, base)
        one16 = jnp.full((16,), 1.0, f32)

        def orow(i, _):
            ones[i // 8, pl.ds((i % 8) * 16, 16)] = one16
            return 0
        lax.fori_loop(0, CH * 8, orow, 0)
        plsc.subcore_barrier()

        def group(g, _):
            pltpu.sync_copy(dst_hbm.at[wid, pl.ds(g * GROUP, GROUP)], didx)

            def chunk(j, _):
                pltpu.sync_copy(ones, acc.at[didx.at[j]], add=True)
                return 0
            lax.fori_loop(0, GROUP, chunk, 0)
            return 0
        lax.fori_loop(0, NGROUP, group, 0)

        plsc.subcore_barrier()
        _writeout_slab(zbuf, acc, out_hbm, cid, base)

    return functools.partial(pl.kernel, out_type=out_type, mesh=mesh,
                             scratch_types=scratch)(body)


_deg = _make_deg()


def _inproj_body(x_ref, w_ref, b_ref, inp_ref, h_ref):
    acc = jnp.dot(x_ref[...], w_ref[...], preferred_element_type=f32) + b_ref[...]
    inp_ref[...] = acc
    h_ref[...] = jnp.maximum(acc, 0.0)


def _inproj(x, W, b):
    return pl.pallas_call(
        _inproj_body,
        grid=(N // BN,),
        in_specs=[
            pl.BlockSpec((BN, D), lambda i: (i, 0)),
            pl.BlockSpec((D, D), lambda i: (0, 0)),
            pl.BlockSpec((1, D), lambda i: (0, 0)),
        ],
        out_specs=[
            pl.BlockSpec((BN, D), lambda i: (i, 0)),
            pl.BlockSpec((BN, D), lambda i: (i, 0)),
        ],
        out_shape=[
            jax.ShapeDtypeStruct((N, D), f32),
            jax.ShapeDtypeStruct((N, D), f32),
        ],
    )(x, W, b.reshape(1, D))


def _conv_body(p0_ref, p1_ref, d0_ref, d1_ref, h_ref, inp_ref, wl_ref, bl_ref,
               wr_ref, out_ref, *, last):
    deg = d0_ref[0][:, 0:1] + d1_ref[0][:, 0:1]
    recip = 1.0 / jnp.maximum(deg, 1.0)
    agg = (p0_ref[0] + p1_ref[0]) * recip
    acc = (jnp.dot(agg, wl_ref[...], preferred_element_type=f32) + bl_ref[...]
           + jnp.dot(h_ref[...], wr_ref[...], preferred_element_type=f32))
    if last:
        m = jnp.max(acc, axis=-1, keepdims=True)
        s = acc - m
        out_ref[...] = s - jnp.log(jnp.sum(jnp.exp(s), axis=-1, keepdims=True))
    else:
        out_ref[...] = jnp.maximum(acc, 0.0) + 0.2 * inp_ref[...]


def _conv(parts, deg, h, inp, Wl, bl, Wr, last):
    return pl.pallas_call(
        functools.partial(_conv_body, last=last),
        grid=(N // BN,),
        in_specs=[
            pl.BlockSpec((1, BN, D), lambda i: (0, i, 0)),
            pl.BlockSpec((1, BN, D), lambda i: (1, i, 0)),
            pl.BlockSpec((1, BN, D), lambda i: (0, i, 0)),
            pl.BlockSpec((1, BN, D), lambda i: (1, i, 0)),
            pl.BlockSpec((BN, D), lambda i: (i, 0)),
            pl.BlockSpec((BN, D), lambda i: (i, 0)),
            pl.BlockSpec((D, D), lambda i: (0, 0)),
            pl.BlockSpec((1, D), lambda i: (0, 0)),
            pl.BlockSpec((D, D), lambda i: (0, 0)),
        ],
        out_specs=pl.BlockSpec((BN, D), lambda i: (i, 0)),
        out_shape=jax.ShapeDtypeStruct((N, D), f32),
    )(parts, parts, deg, deg, h, inp, Wl, bl.reshape(1, D), Wr)


def kernel(x, edge_index, W_in, b_in, Wl0, bl0, Wr0, Wl1, bl1, Wr1, Wl2, bl2, Wr2):
    src = edge_index[0].astype(jnp.int32)
    dst = edge_index[1].astype(jnp.int32)
    srcp = jnp.concatenate([src, jnp.zeros((EP - E,), jnp.int32)])
    dstp = jnp.concatenate([dst, jnp.full((EP - E,), DUMMY, jnp.int32)])
    srcp = srcp.reshape(NW, NCHUNK, CH)
    dstp = dstp.reshape(NW, NCHUNK, CH)

    inp, h0 = _inproj(x, W_in, b_in)
    degp = _deg(dstp)
    parts0 = _agg(h0, srcp, dstp)
    h1 = _conv(parts0, degp, h0, inp, Wl0, bl0, Wr0, last=False)
    parts1 = _agg(h1, srcp, dstp)
    h2 = _conv(parts1, degp, h1, inp, Wl1, bl1, Wr1, last=False)
    parts2 = _agg(h2, srcp, dstp)
    return _conv(parts2, degp, h2, inp, Wl2, bl2, Wr2, last=True)


# speedup vs baseline: 1.1120x; 1.0005x over previous
import functools

import jax
import jax.numpy as jnp
from jax import lax
from jax.experimental import pallas as pl
from jax.experimental.pallas import tpu as pltpu
from jax.experimental.pallas import tpu_sc as plsc

N = 10000
E = 320000
D = 128

NC = 2
NS = 16
NW = NC * NS

CH = 112
EPW = 10752
EP = NW * EPW
NCHUNK = EPW // CH

NP = 10112
RPT = NP // NS
DUMMY = N

BN = 1000

f32 = jnp.float32


GROUP = 96
NGROUP = NCHUNK // GROUP


def _zero_spmem_slab(rows, acc, base):
    z16 = jnp.zeros((16,), f32)

    def zrow(i, _):
        rows[i // 8, pl.ds((i % 8) * 16, 16)] = z16
        return 0
    lax.fori_loop(0, CH * 8, zrow, 0)
    off = 0
    while off < RPT:
        nr = min(CH, RPT - off)
        pltpu.sync_copy(rows.at[pl.ds(0, nr)], acc.at[pl.ds(base + off, nr)])
        off += nr


def _writeout_slab(rows, acc, out_hbm, cid, base):
    off = 0
    while off < RPT:
        nr = min(CH, RPT - off)
        pltpu.sync_copy(acc.at[pl.ds(base + off, nr)], rows.at[pl.ds(0, nr)])
        pltpu.sync_copy(rows.at[pl.ds(0, nr)], out_hbm.at[cid, pl.ds(base + off, nr)])
        off += nr


HROWS = 624


def _make_gather():
    mesh = plsc.VectorSubcoreMesh(core_axis_name="c", subcore_axis_name="s")
    out_type = jax.ShapeDtypeStruct((NW, EPW, D), f32)
    scratch = [
        pltpu.VMEM((GROUP, CH), jnp.int32),
        pltpu.VMEM((CH, D), f32),
        pltpu.VMEM((CH, D), f32),
        pltpu.VMEM_SHARED((N, D), f32),
        pltpu.SemaphoreType.DMA,
        pltpu.SemaphoreType.DMA,
    ]

    def body(h_hbm, src_hbm, msgs_hbm, sidx, rows0, rows1, hsp, sem0, sem1):
        cid = lax.axis_index("c")
        sid = lax.axis_index("s")
        wid = sid * NC + cid
        bufs = (rows0, rows1)
        sems = (sem0, sem1)

        pltpu.sync_copy(h_hbm.at[pl.ds(sid * HROWS, HROWS)],
                        hsp.at[pl.ds(sid * HROWS, HROWS)])

        @pl.when(sid == 0)
        def _():
            pltpu.sync_copy(h_hbm.at[pl.ds(NS * HROWS, N - NS * HROWS)],
                            hsp.at[pl.ds(NS * HROWS, N - NS * HROWS)])
        plsc.subcore_barrier()

        def group(g, _):
            pltpu.sync_copy(src_hbm.at[wid, pl.ds(g * GROUP, GROUP)], sidx)
            gd = [None, None]
            gd[0] = pltpu.async_copy(hsp.at[sidx.at[0]], bufs[0], sems[0])
            for j in range(GROUP):
                b = j % 2
                if j + 1 < GROUP:
                    nb = (j + 1) % 2
                    gd[nb] = pltpu.async_copy(
                        hsp.at[sidx.at[j + 1]], bufs[nb], sems[nb])
                gd[b].wait()
                pltpu.sync_copy(
                    bufs[b], msgs_hbm.at[wid, pl.ds((g * GROUP + j) * CH, CH)])
            return 0
        lax.fori_loop(0, NGROUP, group, 0)

    return functools.partial(pl.kernel, out_type=out_type, mesh=mesh,
                             scratch_types=scratch)(body)


def _make_scatter():
    mesh = plsc.VectorSubcoreMesh(core_axis_name="c", subcore_axis_name="s")
    out_type = jax.ShapeDtypeStruct((NC, NP, D), f32)
    scratch = [
        pltpu.VMEM((GROUP, CH), jnp.int32),
        pltpu.VMEM((CH, D), f32),
        pltpu.VMEM((CH, D), f32),
        pltpu.VMEM_SHARED((NP, D), f32),
        pltpu.SemaphoreType.DMA,
        pltpu.SemaphoreType.DMA,
    ]

    def body(msgs_hbm, dst_hbm, out_hbm, didx, rows0, rows1, acc, sem0, sem1):
        cid = lax.axis_index("c")
        sid = lax.axis_index("s")
        wid = sid * NC + cid
        base = sid * RPT
        bufs = (rows0, rows1)
        sems = (sem0, sem1)

        _zero_spmem_slab(rows0, acc, base)
        plsc.subcore_barrier()

        def group(g, _):
            pltpu.sync_copy(dst_hbm.at[wid, pl.ds(g * GROUP, GROUP)], didx)
            rd = [None, None]
            rd[0] = pltpu.async_copy(
                msgs_hbm.at[wid, pl.ds(g * GROUP * CH, CH)], bufs[0], sems[0])
            for j in range(GROUP):
                b = j % 2
                if j + 1 < GROUP:
                    nb = (j + 1) % 2
                    rd[nb] = pltpu.async_copy(
                        msgs_hbm.at[wid, pl.ds((g * GROUP + j + 1) * CH, CH)],
                        bufs[nb], sems[nb])
                rd[b].wait()
                pltpu.sync_copy(bufs[b], acc.at[didx.at[j]], add=True)
            return 0
        lax.fori_loop(0, NGROUP, group, 0)

        plsc.subcore_barrier()
        _writeout_slab(rows0, acc, out_hbm, cid, base)

    return functools.partial(pl.kernel, out_type=out_type, mesh=mesh,
                             scratch_types=scratch)(body)


_gather_msgs = _make_gather()
_scatter_msgs = _make_scatter()


def _agg(h, srcp, dstp):
    return _scatter_msgs(_gather_msgs(h, srcp), dstp)


def _make_deg():
    mesh = plsc.VectorSubcoreMesh(core_axis_name="c", subcore_axis_name="s")
    out_type = jax.ShapeDtypeStruct((NC, NP, D), f32)
    scratch = [
        pltpu.VMEM((GROUP, CH), jnp.int32),
        pltpu.VMEM((CH, D), f32),
        pltpu.VMEM((CH, D), f32),
        pltpu.VMEM_SHARED((NP, D), f32),
    ]

    def body(dst_hbm, out_hbm, didx, ones, zbuf, acc):
        cid = lax.axis_index("c")
        sid = lax.axis_index("s")
        wid = sid * NC + cid
        base = sid * RPT

        _zero_spmem_slab(zbuf, acc, base)
        one16 = jnp.full((16,), 1.0, f32)

        def orow(i, _):
            ones[i // 8, pl.ds((i % 8) * 16, 16)] = one16
            return 0
        lax.fori_loop(0, CH * 8, orow, 0)
        plsc.subcore_barrier()

        def group(g, _):
            pltpu.sync_copy(dst_hbm.at[wid, pl.ds(g * GROUP, GROUP)], didx)

            def chunk(j, _):
                pltpu.sync_copy(ones, acc.at[didx.at[j]], add=True)
                return 0
            lax.fori_loop(0, GROUP, chunk, 0)
            return 0
        lax.fori_loop(0, NGROUP, group, 0)

        plsc.subcore_barrier()
        _writeout_slab(zbuf, acc, out_hbm, cid, base)

    return functools.partial(pl.kernel, out_type=out_type, mesh=mesh,
                             scratch_types=scratch)(body)


_deg = _make_deg()


def _inproj_body(x_ref, w_ref, b_ref, inp_ref, h_ref):
    acc = jnp.dot(x_ref[...], w_ref[...], preferred_element_type=f32) + b_ref[...]
    inp_ref[...] = acc
    h_ref[...] = jnp.maximum(acc, 0.0)


def _inproj(x, W, b):
    return pl.pallas_call(
        _inproj_body,
        grid=(N // BN,),
        in_specs=[
            pl.BlockSpec((BN, D), lambda i: (i, 0)),
            pl.BlockSpec((D, D), lambda i: (0, 0)),
            pl.BlockSpec((1, D), lambda i: (0, 0)),
        ],
        out_specs=[
            pl.BlockSpec((BN, D), lambda i: (i, 0)),
            pl.BlockSpec((BN, D), lambda i: (i, 0)),
        ],
        out_shape=[
            jax.ShapeDtypeStruct((N, D), f32),
            jax.ShapeDtypeStruct((N, D), f32),
        ],
    )(x, W, b.reshape(1, D))


def _conv_body(p0_ref, p1_ref, d0_ref, d1_ref, h_ref, inp_ref, wl_ref, bl_ref,
               wr_ref, out_ref, *, last):
    deg = d0_ref[0][:, 0:1] + d1_ref[0][:, 0:1]
    recip = 1.0 / jnp.maximum(deg, 1.0)
    agg = (p0_ref[0] + p1_ref[0]) * recip
    acc = (jnp.dot(agg, wl_ref[...], preferred_element_type=f32) + bl_ref[...]
           + jnp.dot(h_ref[...], wr_ref[...], preferred_element_type=f32))
    if last:
        m = jnp.max(acc, axis=-1, keepdims=True)
        s = acc - m
        out_ref[...] = s - jnp.log(jnp.sum(jnp.exp(s), axis=-1, keepdims=True))
    else:
        out_ref[...] = jnp.maximum(acc, 0.0) + 0.2 * inp_ref[...]


def _conv(parts, deg, h, inp, Wl, bl, Wr, last):
    return pl.pallas_call(
        functools.partial(_conv_body, last=last),
        grid=(N // BN,),
        in_specs=[
            pl.BlockSpec((1, BN, D), lambda i: (0, i, 0)),
            pl.BlockSpec((1, BN, D), lambda i: (1, i, 0)),
            pl.BlockSpec((1, BN, D), lambda i: (0, i, 0)),
            pl.BlockSpec((1, BN, D), lambda i: (1, i, 0)),
            pl.BlockSpec((BN, D), lambda i: (i, 0)),
            pl.BlockSpec((BN, D), lambda i: (i, 0)),
            pl.BlockSpec((D, D), lambda i: (0, 0)),
            pl.BlockSpec((1, D), lambda i: (0, 0)),
            pl.BlockSpec((D, D), lambda i: (0, 0)),
        ],
        out_specs=pl.BlockSpec((BN, D), lambda i: (i, 0)),
        out_shape=jax.ShapeDtypeStruct((N, D), f32),
    )(parts, parts, deg, deg, h, inp, Wl, bl.reshape(1, D), Wr)


def kernel(x, edge_index, W_in, b_in, Wl0, bl0, Wr0, Wl1, bl1, Wr1, Wl2, bl2, Wr2):
    src = edge_index[0].astype(jnp.int32)
    dst = edge_index[1].astype(jnp.int32)
    srcp = jnp.concatenate([src, jnp.zeros((EP - E,), jnp.int32)])
    dstp = jnp.concatenate([dst, jnp.full((EP - E,), DUMMY, jnp.int32)])
    srcp = srcp.reshape(NW, NCHUNK, CH)
    dstp = dstp.reshape(NW, NCHUNK, CH)

    inp, h0 = _inproj(x, W_in, b_in)
    degp = _deg(dstp)
    parts0 = _agg(h0, srcp, dstp)
    h1 = _conv(parts0, degp, h0, inp, Wl0, bl0, Wr0, last=False)
    parts1 = _agg(h1, srcp, dstp)
    h2 = _conv(parts1, degp, h1, inp, Wl1, bl1, Wr1, last=False)
    parts2 = _agg(h2, srcp, dstp)
    return _conv(parts2, degp, h2, inp, Wl2, bl2, Wr2, last=True)
